# Initial kernel scaffold; baseline (speedup 1.0000x reference)
import jax
import jax.numpy as jnp
from jax.experimental import pallas as pl


def kernel(selected_activations, selected_indices, k, combination_weights, output_projections, W1, b1, W2, b2):
    raise NotImplementedError("write your pallas kernel here")


# speedup vs baseline: 1.5454x; 1.5454x over previous
import jax
import jax.numpy as jnp
from jax import lax
from jax.experimental import pallas as pl
from jax.experimental.pallas import tpu as pltpu
from jax.experimental.pallas import tpu_sc as plsc

D_MODEL = 1024
N_INPUT = 16384
N_PROCESS = 4096
HIDDEN = 1024
B = 32
S = 8
K_IN = 512
K_OUT = 512

_NS = 16
_GCH = 16


def _trans_body(a_ref, o_ref):
    o_ref[...] = a_ref[...].T


def _trans(cw):
    return pl.pallas_call(
        _trans_body,
        grid=(4, 8),
        in_specs=[pl.BlockSpec((1024, 2048), lambda i, j: (i, j))],
        out_specs=pl.BlockSpec((2048, 1024), lambda i, j: (j, i)),
        out_shape=jax.ShapeDtypeStruct((N_INPUT, N_PROCESS), jnp.float32),
    )(cw)


def _sc_body(cwt_hbm, idx_hbm, sw_out, m_out, idx_v, rows_v, mbuf, sem):
    cid = lax.axis_index("c")
    sid = lax.axis_index("s")
    b = cid * _NS + sid

    pltpu.sync_copy(idx_hbm.at[pl.ds(b * K_IN, K_IN)], idx_v)

    for c in range(K_IN // _GCH):
        pltpu.async_copy(
            cwt_hbm.at[idx_v.at[pl.ds(c * _GCH, _GCH)]], rows_v, sem).wait()
        pltpu.sync_copy(rows_v,
                        sw_out.at[pl.ds(b * K_IN + c * _GCH, _GCH), :])

    def _zero(i, cr):
        mbuf[pl.ds(i * 16, 16)] = jnp.zeros((16,), jnp.float32)
        return cr
    lax.fori_loop(0, N_INPUT // 16, _zero, 0)

    ones = jnp.full((16,), 1.0, jnp.float32)

    def _mask(i, cr):
        plsc.store_scatter(mbuf, [idx_v[pl.ds(i * 16, 16)]], ones)
        return cr
    lax.fori_loop(0, K_IN // 16, _mask, 0)
    pltpu.sync_copy(mbuf, m_out.at[pl.ds(b * N_INPUT, N_INPUT)])


def _gather_stage(cwt, idx_flat):
    mesh = plsc.VectorSubcoreMesh(core_axis_name="c", subcore_axis_name="s",
                                  num_cores=2, num_subcores=_NS)
    return pl.kernel(
        _sc_body,
        out_type=[
            jax.ShapeDtypeStruct((B * K_IN, N_PROCESS), jnp.float32),
            jax.ShapeDtypeStruct((B * N_INPUT,), jnp.float32),
        ],
        mesh=mesh,
        scratch_types=[
            pltpu.VMEM((K_IN,), jnp.int32),
            pltpu.VMEM((_GCH, N_PROCESS), jnp.float32),
            pltpu.VMEM((N_INPUT,), jnp.float32),
            pltpu.SemaphoreType.DMA,
        ],
        compiler_params=pltpu.CompilerParams(needs_layout_passes=False),
    )(cwt, idx_flat)


def _gelu(x):
    return 0.5 * x * (1.0 + lax.erf(x * (2.0 ** -0.5)))


def _einsum_body(a_ref, sw_ref, p_ref):
    p_ref[...] = _gelu(lax.dot_general(
        a_ref[...], sw_ref[...], (((1,), (0,)), ((), ())),
        preferred_element_type=jnp.float32,
        precision=lax.Precision.DEFAULT))


def _einsum(act2d, sw):
    return pl.pallas_call(
        _einsum_body,
        grid=(B,),
        in_specs=[
            pl.BlockSpec((S, K_IN), lambda b: (b, 0)),
            pl.BlockSpec((K_IN, N_PROCESS), lambda b: (b, 0)),
        ],
        out_specs=pl.BlockSpec((S, N_PROCESS), lambda b: (b, 0)),
        out_shape=jax.ShapeDtypeStruct((B * S, N_PROCESS), jnp.float32),
    )(act2d, sw)


_BK2 = 2048


def _mlp1_body(m_ref, w1_ref, b1_ref, h_ref, acc_ref):
    k = pl.program_id(0)

    @pl.when(k == 0)
    def _():
        acc_ref[...] = jnp.zeros_like(acc_ref)

    acc_ref[...] += lax.dot_general(
        m_ref[...], w1_ref[...], (((1,), (0,)), ((), ())),
        preferred_element_type=jnp.float32,
        precision=lax.Precision.DEFAULT)

    @pl.when(k == pl.num_programs(0) - 1)
    def _():
        h_ref[...] = _gelu(acc_ref[...] + b1_ref[...])


def _mlp1(m, w1, b1):
    return pl.pallas_call(
        _mlp1_body,
        grid=(N_INPUT // _BK2,),
        in_specs=[
            pl.BlockSpec((B, _BK2), lambda k: (0, k)),
            pl.BlockSpec((_BK2, HIDDEN), lambda k: (k, 0)),
            pl.BlockSpec((1, HIDDEN), lambda k: (0, 0)),
        ],
        out_specs=pl.BlockSpec((B, HIDDEN), lambda k: (0, 0)),
        out_shape=jax.ShapeDtypeStruct((B, HIDDEN), jnp.float32),
        scratch_shapes=[pltpu.VMEM((B, HIDDEN), jnp.float32)],
    )(m, w1, b1)


def _final_body(p_ref, h_ref, w2_ref, b2_ref, op_ref, out_ref):
    rel = lax.dot_general(
        h_ref[...], w2_ref[...], (((1,), (0,)), ((), ())),
        preferred_element_type=jnp.float32,
        precision=lax.Precision.DEFAULT) + b2_ref[...]
    sig = 1.0 / (1.0 + jnp.exp(-rel))

    p = p_ref[...]
    p3 = p.reshape(B, S, N_PROCESS)
    scores = jnp.max(p3, axis=1) * sig

    bits = lax.bitcast_convert_type(scores, jnp.int32)
    imin = jnp.int32(-2147483648)
    key = jnp.where(bits >= 0, bits, imin - bits)

    def _bit_step(i, t):
        trial = t | lax.shift_left(jnp.int32(1), jnp.int32(31) - i)
        thresh = trial ^ imin
        cnt = jnp.sum((key >= thresh).astype(jnp.int32), axis=1,
                      keepdims=True)
        return jnp.where(cnt >= K_OUT, trial, t)

    t = lax.fori_loop(0, 32, _bit_step, jnp.zeros((B, 1), jnp.int32))
    mask = (key >= (t ^ imin)).astype(jnp.float32)

    pm = (p3 * mask[:, None, :]).reshape(B * S, N_PROCESS)
    out_ref[...] = lax.dot_general(
        pm, op_ref[...], (((1,), (0,)), ((), ())),
        preferred_element_type=jnp.float32,
        precision=lax.Precision.DEFAULT)


def _final(p, h, w2, b2, op):
    return pl.pallas_call(
        _final_body,
        in_specs=[
            pl.BlockSpec((B * S, N_PROCESS), lambda: (0, 0)),
            pl.BlockSpec((B, HIDDEN), lambda: (0, 0)),
            pl.BlockSpec((HIDDEN, N_PROCESS), lambda: (0, 0)),
            pl.BlockSpec((1, N_PROCESS), lambda: (0, 0)),
            pl.BlockSpec((N_PROCESS, D_MODEL), lambda: (0, 0)),
        ],
        out_specs=pl.BlockSpec((B * S, D_MODEL), lambda: (0, 0)),
        out_shape=jax.ShapeDtypeStruct((B * S, D_MODEL), jnp.float32),
    )(p, h, w2, b2, op)


def kernel(selected_activations, selected_indices, k, combination_weights,
           output_projections, W1, b1, W2, b2):
    del k
    act = selected_activations.astype(jnp.float32)
    idx = selected_indices.astype(jnp.int32)

    cwt = _trans(combination_weights)
    sw, m_flat = _gather_stage(cwt, idx.reshape(-1))
    m = m_flat.reshape(B, N_INPUT)

    p = _einsum(act.reshape(B * S, K_IN), sw)
    h = _mlp1(m, W1, b1.reshape(1, HIDDEN))
    out = _final(p, h, W2, b2.reshape(1, N_PROCESS), output_projections)
    return out.reshape(B, S, D_MODEL)


# speedup vs baseline: 1.5938x; 1.0314x over previous
import jax
import jax.numpy as jnp
from jax import lax
from jax.experimental import pallas as pl
from jax.experimental.pallas import tpu as pltpu
from jax.experimental.pallas import tpu_sc as plsc

D_MODEL = 1024
N_INPUT = 16384
N_PROCESS = 4096
HIDDEN = 1024
B = 32
S = 8
K_IN = 512
K_OUT = 512

_NS = 16
_GCH = 8


def _trans_body(a_ref, o_ref):
    o_ref[...] = a_ref[...].T


def _trans(cw):
    return pl.pallas_call(
        _trans_body,
        grid=(4, 8),
        in_specs=[pl.BlockSpec((1024, 2048), lambda i, j: (i, j))],
        out_specs=pl.BlockSpec((2048, 1024), lambda i, j: (j, i)),
        out_shape=jax.ShapeDtypeStruct((N_INPUT, N_PROCESS), jnp.float32),
    )(cw)


def _sc_body(cwt_hbm, idx_hbm, sw_out, m_out, idx_v, rows0, rows1, mbuf,
             sem0, sem1):
    cid = lax.axis_index("c")
    sid = lax.axis_index("s")
    b = cid * _NS + sid

    pltpu.sync_copy(idx_hbm.at[pl.ds(b * K_IN, K_IN)], idx_v)

    bufs = (rows0, rows1)
    sems = (sem0, sem1)
    n_ch = K_IN // _GCH

    def _start(c):
        return pltpu.async_copy(
            cwt_hbm.at[idx_v.at[pl.ds(c * _GCH, _GCH)]],
            bufs[c % 2], sems[c % 2])

    descs = {0: _start(0)}

    def _zero(i, cr):
        mbuf[pl.ds(i * 16, 16)] = jnp.zeros((16,), jnp.float32)
        return cr
    lax.fori_loop(0, N_INPUT // 16, _zero, 0)

    ones = jnp.full((16,), 1.0, jnp.float32)

    def _mask(i, cr):
        plsc.store_scatter(mbuf, [idx_v[pl.ds(i * 16, 16)]], ones)
        return cr
    lax.fori_loop(0, K_IN // 16, _mask, 0)
    pltpu.sync_copy(mbuf, m_out.at[pl.ds(b * N_INPUT, N_INPUT)])

    for c in range(n_ch):
        if c + 1 < n_ch:
            descs[(c + 1) % 2] = _start(c + 1)
        descs[c % 2].wait()
        pltpu.sync_copy(bufs[c % 2],
                        sw_out.at[pl.ds(b * K_IN + c * _GCH, _GCH), :])


def _gather_stage(cwt, idx_flat):
    mesh = plsc.VectorSubcoreMesh(core_axis_name="c", subcore_axis_name="s",
                                  num_cores=2, num_subcores=_NS)
    return pl.kernel(
        _sc_body,
        out_type=[
            jax.ShapeDtypeStruct((B * K_IN, N_PROCESS), jnp.float32),
            jax.ShapeDtypeStruct((B * N_INPUT,), jnp.float32),
        ],
        mesh=mesh,
        scratch_types=[
            pltpu.VMEM((K_IN,), jnp.int32),
            pltpu.VMEM((_GCH, N_PROCESS), jnp.float32),
            pltpu.VMEM((_GCH, N_PROCESS), jnp.float32),
            pltpu.VMEM((N_INPUT,), jnp.float32),
            pltpu.SemaphoreType.DMA,
            pltpu.SemaphoreType.DMA,
        ],
        compiler_params=pltpu.CompilerParams(needs_layout_passes=False),
    )(cwt, idx_flat)


def _gelu(x):
    return 0.5 * x * (1.0 + lax.erf(x * (2.0 ** -0.5)))


def _einsum_body(a_ref, sw_ref, p_ref):
    p_ref[...] = _gelu(lax.dot_general(
        a_ref[...], sw_ref[...], (((1,), (0,)), ((), ())),
        preferred_element_type=jnp.float32,
        precision=lax.Precision.DEFAULT))


def _einsum(act2d, sw):
    return pl.pallas_call(
        _einsum_body,
        grid=(B,),
        in_specs=[
            pl.BlockSpec((S, K_IN), lambda b: (b, 0)),
            pl.BlockSpec((K_IN, N_PROCESS), lambda b: (b, 0)),
        ],
        out_specs=pl.BlockSpec((S, N_PROCESS), lambda b: (b, 0)),
        out_shape=jax.ShapeDtypeStruct((B * S, N_PROCESS), jnp.float32),
    )(act2d, sw)


_BK2 = 2048


def _mlp1_body(m_ref, w1_ref, b1_ref, h_ref, acc_ref):
    k = pl.program_id(0)

    @pl.when(k == 0)
    def _():
        acc_ref[...] = jnp.zeros_like(acc_ref)

    acc_ref[...] += lax.dot_general(
        m_ref[...], w1_ref[...], (((1,), (0,)), ((), ())),
        preferred_element_type=jnp.float32,
        precision=lax.Precision.DEFAULT)

    @pl.when(k == pl.num_programs(0) - 1)
    def _():
        h_ref[...] = _gelu(acc_ref[...] + b1_ref[...])


def _mlp1(m, w1, b1):
    return pl.pallas_call(
        _mlp1_body,
        grid=(N_INPUT // _BK2,),
        in_specs=[
            pl.BlockSpec((B, _BK2), lambda k: (0, k)),
            pl.BlockSpec((_BK2, HIDDEN), lambda k: (k, 0)),
            pl.BlockSpec((1, HIDDEN), lambda k: (0, 0)),
        ],
        out_specs=pl.BlockSpec((B, HIDDEN), lambda k: (0, 0)),
        out_shape=jax.ShapeDtypeStruct((B, HIDDEN), jnp.float32),
        scratch_shapes=[pltpu.VMEM((B, HIDDEN), jnp.float32)],
    )(m, w1, b1)


def _final_body(p_ref, h_ref, w2_ref, b2_ref, op_ref, out_ref):
    rel = lax.dot_general(
        h_ref[...], w2_ref[...], (((1,), (0,)), ((), ())),
        preferred_element_type=jnp.float32,
        precision=lax.Precision.DEFAULT) + b2_ref[...]
    sig = 1.0 / (1.0 + jnp.exp(-rel))

    p = p_ref[...]
    p3 = p.reshape(B, S, N_PROCESS)
    scores = jnp.max(p3, axis=1) * sig

    bits = lax.bitcast_convert_type(scores, jnp.int32)
    imin = jnp.int32(-2147483648)
    key = jnp.where(bits >= 0, bits, imin - bits)

    def _bit_step(i, t):
        trial = t | lax.shift_left(jnp.int32(1), jnp.int32(31) - i)
        thresh = trial ^ imin
        cnt = jnp.sum((key >= thresh).astype(jnp.int32), axis=1,
                      keepdims=True)
        return jnp.where(cnt >= K_OUT, trial, t)

    t = lax.fori_loop(0, 32, _bit_step, jnp.zeros((B, 1), jnp.int32))
    mask = (key >= (t ^ imin)).astype(jnp.float32)

    pm = (p3 * mask[:, None, :]).reshape(B * S, N_PROCESS)
    out_ref[...] = lax.dot_general(
        pm, op_ref[...], (((1,), (0,)), ((), ())),
        preferred_element_type=jnp.float32,
        precision=lax.Precision.DEFAULT)


def _final(p, h, w2, b2, op):
    return pl.pallas_call(
        _final_body,
        in_specs=[
            pl.BlockSpec((B * S, N_PROCESS), lambda: (0, 0)),
            pl.BlockSpec((B, HIDDEN), lambda: (0, 0)),
            pl.BlockSpec((HIDDEN, N_PROCESS), lambda: (0, 0)),
            pl.BlockSpec((1, N_PROCESS), lambda: (0, 0)),
            pl.BlockSpec((N_PROCESS, D_MODEL), lambda: (0, 0)),
        ],
        out_specs=pl.BlockSpec((B * S, D_MODEL), lambda: (0, 0)),
        out_shape=jax.ShapeDtypeStruct((B * S, D_MODEL), jnp.float32),
    )(p, h, w2, b2, op)


def kernel(selected_activations, selected_indices, k, combination_weights,
           output_projections, W1, b1, W2, b2):
    del k
    act = selected_activations.astype(jnp.float32)
    idx = selected_indices.astype(jnp.int32)

    cwt = _trans(combination_weights)
    sw, m_flat = _gather_stage(cwt, idx.reshape(-1))
    m = m_flat.reshape(B, N_INPUT)

    p = _einsum(act.reshape(B * S, K_IN), sw)
    h = _mlp1(m, W1, b1.reshape(1, HIDDEN))
    out = _final(p, h, W2, b2.reshape(1, N_PROCESS), output_projections)
    return out.reshape(B, S, D_MODEL)
